# baseline (speedup 1.0000x reference)
import functools

import jax
import jax.numpy as jnp
from jax.experimental import pallas as pl
from jax.experimental.pallas import tpu as pltpu

N = 4096
D = 512
BI = 512
BJ = 256
NI = N // BI
NJ = N // BJ
NEG_SLOPE = 0.2
NEG_BIG = -1e30


def _lrelu(x):
    return jnp.where(x >= 0, x, NEG_SLOPE * x)


def _proj_body(x_ref, w_ref, asrc_ref, adst_ref, xp_ref, as_ref, ad_ref):
    xp = jax.lax.dot_general(
        x_ref[...], w_ref[...], (((1,), (0,)), ((), ())),
        preferred_element_type=jnp.float32)
    xp_ref[...] = xp
    as_ref[...] = jax.lax.dot_general(
        xp, asrc_ref[...], (((1,), (0,)), ((), ())),
        preferred_element_type=jnp.float32)
    ad_ref[...] = jax.lax.dot_general(
        xp, adst_ref[...], (((1,), (0,)), ((), ())),
        preferred_element_type=jnp.float32)


def _colmax_body(a_ref, as_ref, ad_ref, m_ref):
    i = pl.program_id(1)

    @pl.when(i == 0)
    def _():
        m_ref[...] = jnp.full((1, BJ), NEG_BIG, jnp.float32)

    masked = jnp.where(a_ref[...] != 0.0,
                       jax.lax.broadcast_in_dim(as_ref[...], (BI, BJ), (0, 1)),
                       NEG_BIG)
    m_ref[...] = jnp.maximum(m_ref[...], jnp.max(masked, axis=0, keepdims=True))

    @pl.when(i == NI - 1)
    def _():
        m_ref[...] = _lrelu(m_ref[...] + ad_ref[...])


def _agg_body(a_ref, xp_ref, as_ref, ad_ref, m_ref, bias_ref, out_ref, s_ref):
    i = pl.program_id(1)

    @pl.when(i == 0)
    def _():
        out_ref[...] = jnp.zeros_like(out_ref)
        s_ref[...] = jnp.zeros_like(s_ref)

    e = _lrelu(as_ref[...] + ad_ref[...])
    ex = jnp.where(a_ref[...] != 0.0, jnp.exp(e - m_ref[...]), 0.0)
    xp_blk = xp_ref[pl.ds(i * BI, BI), :]
    out_ref[...] += jax.lax.dot_general(
        ex, xp_blk, (((0,), (0,)), ((), ())),
        preferred_element_type=jnp.float32)
    s_ref[...] += jax.lax.dot_general(
        ex, jnp.ones((BI, 1), jnp.float32), (((0,), (0,)), ((), ())),
        preferred_element_type=jnp.float32)

    @pl.when(i == NI - 1)
    def _():
        s = s_ref[...]
        s_safe = jnp.where(s > 0.0, s, 1.0)
        out_ref[...] = jnp.maximum(out_ref[...] / s_safe + bias_ref[...], 0.0)


@jax.jit
def kernel(A, X, W, a_src, a_dst, bias):
    d_in = X.shape[1]
    xp, as_col, ad_col = pl.pallas_call(
        _proj_body,
        grid=(NI,),
        in_specs=[
            pl.BlockSpec((BI, d_in), lambda i: (i, 0)),
            pl.BlockSpec((d_in, D), lambda i: (0, 0)),
            pl.BlockSpec((D, 1), lambda i: (0, 0)),
            pl.BlockSpec((D, 1), lambda i: (0, 0)),
        ],
        out_specs=[
            pl.BlockSpec((BI, D), lambda i: (i, 0)),
            pl.BlockSpec((BI, 1), lambda i: (i, 0)),
            pl.BlockSpec((BI, 1), lambda i: (i, 0)),
        ],
        out_shape=[
            jax.ShapeDtypeStruct((N, D), jnp.float32),
            jax.ShapeDtypeStruct((N, 1), jnp.float32),
            jax.ShapeDtypeStruct((N, 1), jnp.float32),
        ],
        compiler_params=pltpu.CompilerParams(
            dimension_semantics=("parallel",)),
    )(X, W, a_src.reshape(D, 1), a_dst.reshape(D, 1))

    ad_row = ad_col.reshape(1, N)

    m_row = pl.pallas_call(
        _colmax_body,
        grid=(NJ, NI),
        in_specs=[
            pl.BlockSpec((BI, BJ), lambda j, i: (i, j)),
            pl.BlockSpec((BI, 1), lambda j, i: (i, 0)),
            pl.BlockSpec((1, BJ), lambda j, i: (0, j)),
        ],
        out_specs=pl.BlockSpec((1, BJ), lambda j, i: (0, j)),
        out_shape=jax.ShapeDtypeStruct((1, N), jnp.float32),
        compiler_params=pltpu.CompilerParams(
            dimension_semantics=("parallel", "arbitrary")),
    )(A, as_col, ad_row)

    out = pl.pallas_call(
        _agg_body,
        grid=(NJ, NI),
        in_specs=[
            pl.BlockSpec((BI, BJ), lambda j, i: (i, j)),
            pl.BlockSpec((N, D), lambda j, i: (0, 0)),
            pl.BlockSpec((BI, 1), lambda j, i: (i, 0)),
            pl.BlockSpec((1, BJ), lambda j, i: (0, j)),
            pl.BlockSpec((1, BJ), lambda j, i: (0, j)),
            pl.BlockSpec((1, D), lambda j, i: (0, 0)),
        ],
        out_specs=pl.BlockSpec((BJ, D), lambda j, i: (j, 0)),
        out_shape=jax.ShapeDtypeStruct((N, D), jnp.float32),
        scratch_shapes=[pltpu.VMEM((BJ, 1), jnp.float32)],
        compiler_params=pltpu.CompilerParams(
            dimension_semantics=("parallel", "arbitrary")),
    )(A, xp, as_col, ad_row, m_row, bias.reshape(1, D))

    return out


# speedup vs baseline: 3.1908x; 3.1908x over previous
import jax
import jax.numpy as jnp
from jax.experimental import pallas as pl
from jax.experimental.pallas import tpu as pltpu

N = 4096
D = 512
BJ = 256
NJ = N // BJ
NI_PROJ = 8
NEG_SLOPE = 0.2
LOG2E = 1.4426950408889634


def _lrelu(x):
    return jnp.maximum(x, NEG_SLOPE * x)


def _proj_body(x_ref, w_ref, asrc_ref, adst_ref, xp_ref, as_ref, ad_ref):
    xp = jax.lax.dot_general(
        x_ref[...], w_ref[...], (((1,), (0,)), ((), ())),
        preferred_element_type=jnp.float32)
    xp_ref[...] = xp
    as_ref[...] = jax.lax.dot_general(
        xp, asrc_ref[...], (((1,), (0,)), ((), ())),
        preferred_element_type=jnp.float32)
    ad_ref[...] = jax.lax.dot_general(
        xp, adst_ref[...], (((1,), (0,)), ((), ())),
        preferred_element_type=jnp.float32)


def _agg_body(a_ref, xp_ref, as_ref, ad_ref, m_ref, bias_ref, out_ref):
    z = as_ref[...] + ad_ref[...]
    e2 = _lrelu(z)
    p = a_ref[...] * jnp.exp2(e2 - m_ref[...])
    out = jax.lax.dot_general(
        p, xp_ref[...], (((0,), (0,)), ((), ())),
        preferred_element_type=jnp.float32)
    s = jax.lax.dot_general(
        p, jnp.ones((N, 1), jnp.float32), (((0,), (0,)), ((), ())),
        preferred_element_type=jnp.float32)
    s_safe = jnp.where(s > 0.0, s, 1.0)
    out_ref[...] = jnp.maximum(out / s_safe + bias_ref[...], 0.0)


@jax.jit
def kernel(A, X, W, a_src, a_dst, bias):
    d_in = X.shape[1]
    bi = N // NI_PROJ
    xp, as_col, ad_col = pl.pallas_call(
        _proj_body,
        grid=(NI_PROJ,),
        in_specs=[
            pl.BlockSpec((bi, d_in), lambda i: (i, 0)),
            pl.BlockSpec((d_in, D), lambda i: (0, 0)),
            pl.BlockSpec((D, 1), lambda i: (0, 0)),
            pl.BlockSpec((D, 1), lambda i: (0, 0)),
        ],
        out_specs=[
            pl.BlockSpec((bi, D), lambda i: (i, 0)),
            pl.BlockSpec((bi, 1), lambda i: (i, 0)),
            pl.BlockSpec((bi, 1), lambda i: (i, 0)),
        ],
        out_shape=[
            jax.ShapeDtypeStruct((N, D), jnp.float32),
            jax.ShapeDtypeStruct((N, 1), jnp.float32),
            jax.ShapeDtypeStruct((N, 1), jnp.float32),
        ],
        compiler_params=pltpu.CompilerParams(
            dimension_semantics=("parallel",)),
    )(X, W, (a_src * LOG2E).reshape(D, 1), (a_dst * LOG2E).reshape(D, 1))

    ad_row = ad_col.reshape(1, N)
    q = jnp.max(as_col) + ad_row
    m_row = jnp.maximum(q, NEG_SLOPE * q)

    out = pl.pallas_call(
        _agg_body,
        grid=(NJ,),
        in_specs=[
            pl.BlockSpec((N, BJ), lambda j: (0, j)),
            pl.BlockSpec((N, D), lambda j: (0, 0)),
            pl.BlockSpec((N, 1), lambda j: (0, 0)),
            pl.BlockSpec((1, BJ), lambda j: (0, j)),
            pl.BlockSpec((1, BJ), lambda j: (0, j)),
            pl.BlockSpec((1, D), lambda j: (0, 0)),
        ],
        out_specs=pl.BlockSpec((BJ, D), lambda j: (j, 0)),
        out_shape=jax.ShapeDtypeStruct((N, D), jnp.float32),
        compiler_params=pltpu.CompilerParams(
            dimension_semantics=("arbitrary",)),
    )(A, xp, as_col, ad_row, m_row, bias.reshape(1, D))

    return out
